# baseline (speedup 1.0000x reference)
import jax
import jax.numpy as jnp
from jax import lax
from jax.experimental import pallas as pl
from jax.experimental.pallas import tpu as pltpu
from jax.experimental.pallas import tpu_sc as plsc

VALUE_MIN_ = -1.0
VALUE_MAX_ = 1.0
ATOMS_ = 128
LANES = 16


def _build_sc_call(n):
    info = plsc.get_sparse_core_info()
    nc, ns = info.num_cores, info.num_subcores
    nw = nc * ns
    rows_per_w = n // nw
    chunk = 128 if rows_per_w % 512 == 0 else rows_per_w
    nchunk = rows_per_w // chunk
    nbuf = 4
    groups = chunk // LANES
    const_norm = (ATOMS_ - 1) / (VALUE_MAX_ - VALUE_MIN_)

    mesh = plsc.VectorSubcoreMesh(core_axis_name="c", subcore_axis_name="s")

    def body(value_hbm, out_hbm, vals, bufs, idxs, sems):
        cid = lax.axis_index("c")
        sid = lax.axis_index("s")
        wid = sid * nc + cid
        row0 = wid * rows_per_w
        in_copy = pltpu.make_async_copy(
            value_hbm.at[pl.ds(row0, rows_per_w)], vals, sems[0]
        )
        in_copy.start()

        iota = lax.iota(jnp.int32, LANES)
        zeros16 = jnp.zeros((LANES,), jnp.float32)

        def zero_buf(buf):
            def zbody(k, carry):
                base = k * (8 * LANES)
                for j in range(8):
                    buf[pl.ds(base + j * LANES, LANES)] = zeros16
                return carry

            lax.fori_loop(0, chunk * ATOMS_ // (8 * LANES), zbody, 0)

        zero_buf(bufs[0])
        in_copy.wait()

        def bins(c, g):
            v = vals[pl.ds(c * chunk + g * LANES, LANES)]
            v = jnp.clip(v, VALUE_MIN_, VALUE_MAX_)
            vn = (v - VALUE_MIN_) * const_norm
            vn = jnp.clip(vn, 0.0, float(ATOMS_ - 1))
            lo = vn.astype(jnp.int32)
            frac = vn - lo.astype(jnp.float32)
            up = jnp.minimum(lo + 1, ATOMS_ - 1)
            fb = (g * LANES + iota) * ATOMS_
            return fb + lo, fb + up, frac

        unroll = 1

        def write_chunk(c, buf, idx):
            def qbody(q, carry):
                for j in range(unroll):
                    g = q * unroll + j
                    ilo, iup, frac = bins(c, g)
                    plsc.store_scatter(buf, [ilo], 1.0 - frac)
                    plsc.addupdate_scatter(buf, [iup], frac)
                    idx[pl.ds(g * LANES, LANES)] = ilo
                    idx[pl.ds(chunk + g * LANES, LANES)] = iup
                return carry

            lax.fori_loop(0, groups // unroll, qbody, 0)

        def zero_chunk(buf, idx):
            def qbody(q, carry):
                for j in range(unroll):
                    g = q * unroll + j
                    plsc.store_scatter(buf, [idx[pl.ds(g * LANES, LANES)]], zeros16)
                    plsc.store_scatter(buf, [idx[pl.ds(chunk + g * LANES, LANES)]], zeros16)
                return carry

            lax.fori_loop(0, groups // unroll, qbody, 0)

        def dma(c, buf, sem):
            dst = out_hbm.at[pl.ds((row0 + c * chunk) * ATOMS_, chunk * ATOMS_)]
            return pltpu.make_async_copy(buf, dst, sem)

        for b in range(1, nbuf):
            zero_buf(bufs[b])

        def cbody(t, carry):
            for b in range(nbuf):
                c = nbuf * t + b

                @pl.when(t > 0)
                def _wait_and_clean():
                    dma(c - nbuf, bufs[b], sems[b]).wait()
                    zero_chunk(bufs[b], idxs[b])

                write_chunk(c, bufs[b], idxs[b])
                dma(c, bufs[b], sems[b]).start()
            return carry

        lax.fori_loop(0, nchunk // nbuf, cbody, 0)
        for b in range(nbuf):
            dma(nchunk - nbuf + b, bufs[b], sems[b]).wait()

    return pl.kernel(
        body,
        out_type=jax.ShapeDtypeStruct((n * ATOMS_,), jnp.float32),
        mesh=mesh,
        scratch_types=[
            pltpu.VMEM((rows_per_w,), jnp.float32),
            [pltpu.VMEM((chunk * ATOMS_,), jnp.float32) for _ in range(nbuf)],
            [pltpu.VMEM((2 * chunk,), jnp.int32) for _ in range(nbuf)],
            [pltpu.SemaphoreType.DMA for _ in range(nbuf)],
        ],
        compiler_params=pltpu.CompilerParams(needs_layout_passes=False),
    )


@jax.jit
def kernel(value):
    n = value.shape[0]
    out = _build_sc_call(n)(value.reshape(n))
    return out.reshape(n, ATOMS_)


# speedup vs baseline: 1.0313x; 1.0313x over previous
import jax
import jax.numpy as jnp
from jax import lax
from jax.experimental import pallas as pl
from jax.experimental.pallas import tpu as pltpu
from jax.experimental.pallas import tpu_sc as plsc

VALUE_MIN_ = -1.0
VALUE_MAX_ = 1.0
ATOMS_ = 128
LANES = 16


def _build_sc_call(n):
    info = plsc.get_sparse_core_info()
    nc, ns = info.num_cores, info.num_subcores
    nw = nc * ns
    rows_per_w = n // nw
    chunk = 128 if rows_per_w % 512 == 0 else rows_per_w
    nchunk = rows_per_w // chunk
    nbuf = 4
    groups = chunk // LANES
    const_norm = (ATOMS_ - 1) / (VALUE_MAX_ - VALUE_MIN_)

    mesh = plsc.VectorSubcoreMesh(core_axis_name="c", subcore_axis_name="s")

    def body(value_hbm, out_hbm, vals, bufs, idxs, sems):
        cid = lax.axis_index("c")
        sid = lax.axis_index("s")
        wid = sid * nc + cid
        row0 = wid * rows_per_w
        in_copy = pltpu.make_async_copy(
            value_hbm.at[pl.ds(row0, rows_per_w)], vals, sems[0]
        )
        in_copy.start()

        iota = lax.iota(jnp.int32, LANES)
        zeros16 = jnp.zeros((LANES,), jnp.float32)

        def zero_buf(buf):
            def zbody(k, carry):
                base = k * (8 * LANES)
                for j in range(8):
                    buf[pl.ds(base + j * LANES, LANES)] = zeros16
                return carry

            lax.fori_loop(0, chunk * ATOMS_ // (8 * LANES), zbody, 0)

        zero_buf(bufs[0])
        in_copy.wait()

        def bins(c, g):
            v = vals[pl.ds(c * chunk + g * LANES, LANES)]
            v = jnp.clip(v, VALUE_MIN_, VALUE_MAX_)
            vn = (v - VALUE_MIN_) * const_norm
            vn = jnp.clip(vn, 0.0, float(ATOMS_ - 1))
            lo = vn.astype(jnp.int32)
            frac = vn - lo.astype(jnp.float32)
            up = jnp.minimum(lo + 1, ATOMS_ - 1)
            fb = (g * LANES + iota) * ATOMS_
            return fb + lo, fb + up, frac

        unroll = 1

        def write_chunk(c, buf, idx):
            def qbody(q, carry):
                for j in range(unroll):
                    g = q * unroll + j
                    ilo, iup, frac = bins(c, g)
                    plsc.store_scatter(buf, [ilo], 1.0 - frac)
                    plsc.addupdate_scatter(buf, [iup], frac)
                    idx[pl.ds(g * LANES, LANES)] = ilo
                    idx[pl.ds(chunk + g * LANES, LANES)] = iup
                return carry

            lax.fori_loop(0, groups // unroll, qbody, 0)

        def zero_chunk(buf, idx):
            def qbody(q, carry):
                for j in range(unroll):
                    g = q * unroll + j
                    plsc.store_scatter(buf, [idx[pl.ds(g * LANES, LANES)]], zeros16)
                    plsc.store_scatter(buf, [idx[pl.ds(chunk + g * LANES, LANES)]], zeros16)
                return carry

            lax.fori_loop(0, groups // unroll, qbody, 0)

        def dma(c, buf, sem):
            dst = out_hbm.at[pl.ds((row0 + c * chunk) * ATOMS_, chunk * ATOMS_)]
            return pltpu.make_async_copy(buf, dst, sem)

        for b in range(nbuf):
            if b > 0:
                zero_buf(bufs[b])
            write_chunk(b, bufs[b], idxs[b])
            dma(b, bufs[b], sems[b]).start()

        def cbody(t, carry):
            for b in range(nbuf):
                c = nbuf * t + b
                dma(c - nbuf, bufs[b], sems[b]).wait()
                zero_chunk(bufs[b], idxs[b])
                write_chunk(c, bufs[b], idxs[b])
                dma(c, bufs[b], sems[b]).start()
            return carry

        lax.fori_loop(1, nchunk // nbuf, cbody, 0)
        for b in range(nbuf):
            dma(nchunk - nbuf + b, bufs[b], sems[b]).wait()

    return pl.kernel(
        body,
        out_type=jax.ShapeDtypeStruct((n * ATOMS_,), jnp.float32),
        mesh=mesh,
        scratch_types=[
            pltpu.VMEM((rows_per_w,), jnp.float32),
            [pltpu.VMEM((chunk * ATOMS_,), jnp.float32) for _ in range(nbuf)],
            [pltpu.VMEM((2 * chunk,), jnp.int32) for _ in range(nbuf)],
            [pltpu.SemaphoreType.DMA for _ in range(nbuf)],
        ],
        compiler_params=pltpu.CompilerParams(needs_layout_passes=False),
    )


@jax.jit
def kernel(value):
    n = value.shape[0]
    out = _build_sc_call(n)(value.reshape(n))
    return out.reshape(n, ATOMS_)


# speedup vs baseline: 1.0324x; 1.0010x over previous
import jax
import jax.numpy as jnp
from jax import lax
from jax.experimental import pallas as pl
from jax.experimental.pallas import tpu as pltpu
from jax.experimental.pallas import tpu_sc as plsc

VALUE_MIN_ = -1.0
VALUE_MAX_ = 1.0
ATOMS_ = 128
LANES = 16


def _build_sc_call(n):
    info = plsc.get_sparse_core_info()
    nc, ns = info.num_cores, info.num_subcores
    nw = nc * ns
    rows_per_w = n // nw
    chunk = 128 if rows_per_w % 512 == 0 else rows_per_w
    nchunk = rows_per_w // chunk
    nbuf = 4
    groups = chunk // LANES
    const_norm = (ATOMS_ - 1) / (VALUE_MAX_ - VALUE_MIN_)

    mesh = plsc.VectorSubcoreMesh(core_axis_name="c", subcore_axis_name="s")

    def body(value_hbm, out_hbm, vals, bufs, idxs, sems):
        cid = lax.axis_index("c")
        sid = lax.axis_index("s")
        wid = sid * nc + cid
        row0 = wid * rows_per_w
        in_copy = pltpu.make_async_copy(
            value_hbm.at[pl.ds(row0, rows_per_w)], vals, sems[0]
        )
        in_copy.start()

        iota = lax.iota(jnp.int32, LANES)
        zeros16 = jnp.zeros((LANES,), jnp.float32)

        def zero_buf(buf):
            def zbody(k, carry):
                base = k * (8 * LANES)
                for j in range(8):
                    buf[pl.ds(base + j * LANES, LANES)] = zeros16
                return carry

            lax.fori_loop(0, chunk * ATOMS_ // (8 * LANES), zbody, 0)

        zero_buf(bufs[0])
        in_copy.wait()

        def bins(c, g):
            v = vals[pl.ds(c * chunk + g * LANES, LANES)]
            v = jnp.clip(v, VALUE_MIN_, VALUE_MAX_)
            vn = (v - VALUE_MIN_) * const_norm
            vn = jnp.clip(vn, 0.0, float(ATOMS_ - 1))
            lo = vn.astype(jnp.int32)
            frac = vn - lo.astype(jnp.float32)
            up = jnp.minimum(lo + 1, ATOMS_ - 1)
            fb = (g * LANES + iota) * ATOMS_
            return fb + lo, fb + up, frac

        def write_chunk(c, buf, idx):
            def gbody(g, carry):
                ilo, iup, frac = bins(c, g)
                plsc.store_scatter(buf, [ilo], 1.0 - frac)
                plsc.addupdate_scatter(buf, [iup], frac)
                idx[pl.ds(g * LANES, LANES)] = ilo
                idx[pl.ds(chunk + g * LANES, LANES)] = iup
                return carry

            lax.fori_loop(0, groups, gbody, 0)

        def zero_chunk(buf, idx):
            def gbody(g, carry):
                plsc.store_scatter(buf, [idx[pl.ds(g * LANES, LANES)]], zeros16)
                plsc.store_scatter(buf, [idx[pl.ds(chunk + g * LANES, LANES)]], zeros16)
                return carry

            lax.fori_loop(0, groups, gbody, 0)

        def dma(c, buf, sem):
            dst = out_hbm.at[pl.ds((row0 + c * chunk) * ATOMS_, chunk * ATOMS_)]
            return pltpu.make_async_copy(buf, dst, sem)

        for b in range(nbuf):
            if b > 0:
                zero_buf(bufs[b])
            write_chunk(b, bufs[b], idxs[b])
            dma(b, bufs[b], sems[b]).start()

        def cbody(t, carry):
            for b in range(nbuf):
                c = nbuf * t + b
                dma(c - nbuf, bufs[b], sems[b]).wait()
                zero_chunk(bufs[b], idxs[b])
                write_chunk(c, bufs[b], idxs[b])
                dma(c, bufs[b], sems[b]).start()
            return carry

        lax.fori_loop(1, nchunk // nbuf, cbody, 0)
        for b in range(nbuf):
            dma(nchunk - nbuf + b, bufs[b], sems[b]).wait()

    return pl.kernel(
        body,
        out_type=jax.ShapeDtypeStruct((n * ATOMS_,), jnp.float32),
        mesh=mesh,
        scratch_types=[
            pltpu.VMEM((rows_per_w,), jnp.float32),
            [pltpu.VMEM((chunk * ATOMS_,), jnp.float32) for _ in range(nbuf)],
            [pltpu.VMEM((2 * chunk,), jnp.int32) for _ in range(nbuf)],
            [pltpu.SemaphoreType.DMA for _ in range(nbuf)],
        ],
        compiler_params=pltpu.CompilerParams(needs_layout_passes=False),
    )


@jax.jit
def kernel(value):
    n = value.shape[0]
    out = _build_sc_call(n)(value.reshape(n))
    return out.reshape(n, ATOMS_)
